# baseline (speedup 1.0000x reference)
import functools

import jax
import jax.numpy as jnp
from jax import lax
from jax.experimental import pallas as pl
from jax.experimental.pallas import tpu as pltpu
from jax.experimental.pallas import tpu_sc as plsc

N = 50000
E = 800000
IN_F = 1433
HID = 384
EMB = 128
NCLS = 7

NC, NS = 2, 16
NW = NC * NS
LANES = 16

MPAD = 50176
KPAD = 1536
NB = 51200
NBINS = 392
NBINS_PAD = 400
BINSZ = 128
SENT = N
EP = 802944

W_FULL = 6250
W_BASE = W_FULL // NW
W_EXTRA = W_FULL - W_BASE * NW

_SC_MESH = dict(core_axis_name="c", subcore_axis_name="s")
_SC_PARAMS = pltpu.CompilerParams(needs_layout_passes=False)


def _worker_id():
    return lax.axis_index("s") * NC + lax.axis_index("c")


def _p1_body(src_h, dst_h, cnt_out, hist_out,
             sbuf, dbuf, ones_v, zbuf, outv, hs_src, hs_dst, cnt_s):
    cid = lax.axis_index("c")
    sid = lax.axis_index("s")
    wid = _worker_id()

    zt = NB // NS
    zv = jnp.zeros((LANES,), jnp.float32)

    def zb(i, c):
        zbuf[pl.ds(i * LANES, LANES)] = zv
        return c
    lax.fori_loop(0, zt // LANES, zb, 0)
    pltpu.sync_copy(zbuf, hs_src.at[pl.ds(sid * zt, zt)])
    pltpu.sync_copy(zbuf, hs_dst.at[pl.ds(sid * zt, zt)])

    def zc(i, c):
        cnt_s[i] = 0
        return c
    lax.fori_loop(0, NBINS_PAD, zc, 0)
    plsc.subcore_barrier()

    def ob(i, c):
        ones_v[pl.ds(i * LANES, LANES)] = jnp.ones((LANES,), jnp.float32)
        return c
    lax.fori_loop(0, BINSZ // LANES, ob, 0)

    nw = W_BASE + jnp.where(wid < W_EXTRA, 1, 0)
    wbase = wid * W_BASE + jnp.minimum(wid, W_EXTRA)

    def window(j, c):
        base = pl.multiple_of((wbase + j) * 128, 128)
        pltpu.sync_copy(src_h.at[pl.ds(base, 128)], sbuf)
        pltpu.sync_copy(dst_h.at[pl.ds(base, 128)], dbuf)
        pltpu.sync_copy(ones_v, hs_src.at[sbuf], add=True)
        pltpu.sync_copy(ones_v, hs_dst.at[dbuf], add=True)
        for g in range(8):
            dvec = dbuf[pl.ds(g * LANES, LANES)]
            bins = lax.shift_right_logical(dvec, 7)
            for l in range(LANES):
                b = bins[l]
                cnt_s[b] = cnt_s[b] + 1
        return c
    lax.fori_loop(0, nw, window, 0)

    lanes = lax.iota(jnp.int32, LANES)

    def cb(v, c):
        vec = jnp.zeros((LANES,), jnp.int32)
        for l in range(LANES):
            vec = jnp.where(lanes == l, cnt_s[v * LANES + l], vec)
        outv[pl.ds(v * LANES, LANES)] = vec
        return c
    lax.fori_loop(0, NBINS_PAD // LANES, cb, 0)
    pltpu.sync_copy(outv, cnt_out.at[wid])

    plsc.subcore_barrier()
    pltpu.sync_copy(hs_src.at[pl.ds(sid * zt, zt)],
                    hist_out.at[cid * 2].at[pl.ds(sid * zt, zt)])
    pltpu.sync_copy(hs_dst.at[pl.ds(sid * zt, zt)],
                    hist_out.at[cid * 2 + 1].at[pl.ds(sid * zt, zt)])


def _p1(src, dst):
    k = pl.kernel(
        _p1_body,
        out_type=(jax.ShapeDtypeStruct((NW, NBINS_PAD), jnp.int32),
                  jax.ShapeDtypeStruct((4, NB), jnp.float32)),
        mesh=plsc.VectorSubcoreMesh(**_SC_MESH),
        compiler_params=_SC_PARAMS,
        scratch_types=[
            pltpu.VMEM((128,), jnp.int32),
            pltpu.VMEM((128,), jnp.int32),
            pltpu.VMEM((128,), jnp.float32),
            pltpu.VMEM((NB // NS,), jnp.float32),
            pltpu.VMEM((NBINS_PAD,), jnp.int32),
            pltpu.VMEM_SHARED((NB,), jnp.float32),
            pltpu.VMEM_SHARED((NB,), jnp.float32),
            pltpu.SMEM((NBINS_PAD,), jnp.int32),
        ],
    )
    return k(src, dst)


def _p2_body(src_h, dst_h, cnt_h, srcb, dstb, starts_o, tcnt_o,
             cnt_v, s_v, t_v, sbuf, dbuf, pos_v, dstage, off_s):
    wid = _worker_id()
    pltpu.sync_copy(cnt_h, cnt_v)

    lanes = lax.iota(jnp.int32, LANES)
    zi = jnp.zeros((LANES,), jnp.int32)

    def pf(v, carry):
        colsum = zi
        mine = zi
        for w in range(NW):
            cw = cnt_v[w, pl.ds(v * LANES, LANES)]
            colsum = colsum + cw
            mine = mine + jnp.where(w < wid, cw, zi)
        t8 = (colsum + 7) & ~7
        cs = plsc.cumsum(t8)
        s_vec = cs - t8 + carry
        t_v[pl.ds(v * LANES, LANES)] = colsum
        s_v[pl.ds(v * LANES, LANES)] = s_vec
        myoff = s_vec + mine
        for l in range(LANES):
            off_s[v * LANES + l] = myoff[l]
        return carry + cs[15]
    lax.fori_loop(0, NBINS_PAD // LANES, pf, 0)

    @pl.when(wid == 0)
    def _():
        pltpu.sync_copy(s_v, starts_o)
        pltpu.sync_copy(t_v, tcnt_o)

    nw = W_BASE + jnp.where(wid < W_EXTRA, 1, 0)
    wbase = wid * W_BASE + jnp.minimum(wid, W_EXTRA)

    def window(j, c):
        base = pl.multiple_of((wbase + j) * 128, 128)
        pltpu.sync_copy(src_h.at[pl.ds(base, 128)], sbuf)
        pltpu.sync_copy(dst_h.at[pl.ds(base, 128)], dbuf)
        for g in range(8):
            dvec = dbuf[pl.ds(g * LANES, LANES)]
            bins = lax.shift_right_logical(dvec, 7)
            pos = zi
            for l in range(LANES):
                b = bins[l]
                p = off_s[b]
                off_s[b] = p + 1
                pos = jnp.where(lanes == l, p, pos)
            pos_v[pl.ds(g * LANES, LANES)] = pos
            dstage[pl.ds(g * LANES, LANES)] = dvec & 127
        pltpu.sync_copy(sbuf, srcb.at[pos_v])
        pltpu.sync_copy(dstage, dstb.at[pos_v])
        return c
    lax.fori_loop(0, nw, window, 0)


def _p2(src, dst, cnts):
    k = pl.kernel(
        _p2_body,
        out_type=(jax.ShapeDtypeStruct((EP,), jnp.int32),
                  jax.ShapeDtypeStruct((EP,), jnp.int32),
                  jax.ShapeDtypeStruct((NBINS_PAD,), jnp.int32),
                  jax.ShapeDtypeStruct((NBINS_PAD,), jnp.int32)),
        mesh=plsc.VectorSubcoreMesh(**_SC_MESH),
        compiler_params=_SC_PARAMS,
        scratch_types=[
            pltpu.VMEM((NW, NBINS_PAD), jnp.int32),
            pltpu.VMEM((NBINS_PAD,), jnp.int32),
            pltpu.VMEM((NBINS_PAD,), jnp.int32),
            pltpu.VMEM((128,), jnp.int32),
            pltpu.VMEM((128,), jnp.int32),
            pltpu.VMEM((128,), jnp.int32),
            pltpu.VMEM((128,), jnp.int32),
            pltpu.SMEM((NBINS_PAD,), jnp.int32),
        ],
    )
    return k(src, dst, cnts)


def _p3_body(F, h_hbm, srcb, dstb, starts_h, tcnt_h, out_hbm,
             sv_v, tv_v, si_v, di_v, rows_v, acc_v, sem, st_s, tc_s):
    wid = _worker_id()
    pltpu.sync_copy(starts_h, sv_v)
    pltpu.sync_copy(tcnt_h, tv_v)

    def ld(v, c):
        a = sv_v[pl.ds(v * LANES, LANES)]
        b = tv_v[pl.ds(v * LANES, LANES)]
        for l in range(LANES):
            st_s[v * LANES + l] = a[l]
            tc_s[v * LANES + l] = b[l]
        return c
    lax.fori_loop(0, NBINS_PAD // LANES, ld, 0)

    lanes = lax.iota(jnp.int32, LANES)
    CG = F // LANES
    zv = jnp.zeros((LANES,), jnp.float32)

    def perbin(jbin, cb):
        bin_ = jbin * NW + wid

        @pl.when(bin_ < NBINS)
        def _():
            def zrow(r, c):
                for cc in range(CG):
                    acc_v[r, pl.ds(cc * LANES, LANES)] = zv
                return c
            lax.fori_loop(0, BINSZ, zrow, 0)

            n = tc_s[bin_]
            start = pl.multiple_of(st_s[bin_], 8)
            nwin = (n + 63) // 64

            def window(w, c):
                base = start + w * 64
                pltpu.sync_copy(srcb.at[pl.ds(base, 64)], si_v)
                pltpu.sync_copy(dstb.at[pl.ds(base, 64)], di_v)
                rem = n - w * 64
                for g in range(4):
                    valid = (lanes + g * LANES) < rem
                    siv = si_v[pl.ds(g * LANES, LANES)]
                    si_v[pl.ds(g * LANES, LANES)] = jnp.where(valid, siv, SENT)
                    div = di_v[pl.ds(g * LANES, LANES)]
                    di_v[pl.ds(g * LANES, LANES)] = jnp.where(valid, div, 0)
                pltpu.async_copy(h_hbm.at[si_v], rows_v, sem).wait()

                def grp(g, c2):
                    dvec = di_v[pl.ds(g * LANES, LANES)]
                    for l in range(LANES):
                        dl = dvec[l]
                        e = g * LANES + l
                        for cc in range(CG):
                            acc_v[dl, pl.ds(cc * LANES, LANES)] = (
                                acc_v[dl, pl.ds(cc * LANES, LANES)]
                                + rows_v[e, pl.ds(cc * LANES, LANES)])
                    return c2
                lax.fori_loop(0, 4, grp, 0)
                return c
            lax.fori_loop(0, nwin, window, 0)
            pltpu.sync_copy(acc_v, out_hbm.at[pl.ds(bin_ * BINSZ, BINSZ)])
        return cb
    lax.fori_loop(0, 13, perbin, 0)


def _p3(h, srcb, dstb, starts, tcnt, F):
    k = pl.kernel(
        functools.partial(_p3_body, F),
        out_type=jax.ShapeDtypeStruct((MPAD, F), jnp.float32),
        mesh=plsc.VectorSubcoreMesh(**_SC_MESH),
        compiler_params=_SC_PARAMS,
        scratch_types=[
            pltpu.VMEM((NBINS_PAD,), jnp.int32),
            pltpu.VMEM((NBINS_PAD,), jnp.int32),
            pltpu.VMEM((64,), jnp.int32),
            pltpu.VMEM((64,), jnp.int32),
            pltpu.VMEM((64, F), jnp.float32),
            pltpu.VMEM((BINSZ, F), jnp.float32),
            pltpu.SemaphoreType.DMA,
            pltpu.SMEM((NBINS_PAD,), jnp.int32),
            pltpu.SMEM((NBINS_PAD,), jnp.int32),
        ],
    )
    return k(h, srcb, dstb, starts, tcnt)


MB = 1024
NMB = MPAD // MB


def _norm(d0, d1):
    return lax.rsqrt(jnp.maximum(d0 + d1, 1.0))


def _mm1_body(x_ref, w_ref, d0_ref, d1_ref, o_ref):
    acc = jnp.dot(x_ref[...], w_ref[...], preferred_element_type=jnp.float32)
    o_ref[...] = acc * _norm(d0_ref[...], d1_ref[...])


def _mm1(xp, w1p, do0, do1):
    return pl.pallas_call(
        _mm1_body,
        grid=(NMB,),
        in_specs=[
            pl.BlockSpec((MB, KPAD), lambda i: (i, 0)),
            pl.BlockSpec((KPAD, HID), lambda i: (0, 0)),
            pl.BlockSpec((MB, 1), lambda i: (i, 0)),
            pl.BlockSpec((MB, 1), lambda i: (i, 0)),
        ],
        out_specs=pl.BlockSpec((MB, HID), lambda i: (i, 0)),
        out_shape=jax.ShapeDtypeStruct((MPAD, HID), jnp.float32),
    )(xp, w1p, do0, do1)


def _mm2_body(s1_ref, w2_ref, b1_ref, di0_ref, di1_ref, do0_ref, do1_ref,
              o_ref):
    i = pl.program_id(0)
    nin = _norm(di0_ref[...], di1_ref[...])
    h1 = jnp.maximum(s1_ref[...] * nin + b1_ref[0:1, :], 0.0)
    g = jnp.dot(h1, w2_ref[...], preferred_element_type=jnp.float32)
    g = g * _norm(do0_ref[...], do1_ref[...])
    rid = lax.broadcasted_iota(jnp.int32, (MB, EMB), 0) + i * MB
    o_ref[...] = jnp.where(rid < N, g, 0.0)


def _mm2(s1, w2, b1t, di0, di1, do0, do1):
    return pl.pallas_call(
        _mm2_body,
        grid=(NMB,),
        in_specs=[
            pl.BlockSpec((MB, HID), lambda i: (i, 0)),
            pl.BlockSpec((HID, EMB), lambda i: (0, 0)),
            pl.BlockSpec((8, HID), lambda i: (0, 0)),
            pl.BlockSpec((MB, 1), lambda i: (i, 0)),
            pl.BlockSpec((MB, 1), lambda i: (i, 0)),
            pl.BlockSpec((MB, 1), lambda i: (i, 0)),
            pl.BlockSpec((MB, 1), lambda i: (i, 0)),
        ],
        out_specs=pl.BlockSpec((MB, EMB), lambda i: (i, 0)),
        out_shape=jax.ShapeDtypeStruct((MPAD, EMB), jnp.float32),
    )(s1, w2, b1t, di0, di1, do0, do1)


def _mm3_body(s2_ref, wc_ref, b2_ref, bc_ref, di0_ref, di1_ref, o_ref):
    nin = _norm(di0_ref[...], di1_ref[...])
    h2 = s2_ref[...] * nin + b2_ref[0:1, :]
    o_ref[...] = (jnp.dot(h2, wc_ref[...], preferred_element_type=jnp.float32)
                  + bc_ref[0:1, :])


def _mm3(s2, wcp, b2t, bct, di0, di1):
    return pl.pallas_call(
        _mm3_body,
        grid=(NMB,),
        in_specs=[
            pl.BlockSpec((MB, EMB), lambda i: (i, 0)),
            pl.BlockSpec((EMB, 128), lambda i: (0, 0)),
            pl.BlockSpec((8, EMB), lambda i: (0, 0)),
            pl.BlockSpec((8, 128), lambda i: (0, 0)),
            pl.BlockSpec((MB, 1), lambda i: (i, 0)),
            pl.BlockSpec((MB, 1), lambda i: (i, 0)),
        ],
        out_specs=pl.BlockSpec((MB, 128), lambda i: (i, 0)),
        out_shape=jax.ShapeDtypeStruct((MPAD, 128), jnp.float32),
    )(s2, wcp, b2t, bct, di0, di1)


def kernel(graph, features, W1, b1, W2, b2, Wc, bc):
    src = graph[0]
    dst = graph[1]

    cnts, hists = _p1(src, dst)
    srcb, dstb, starts, tcnt = _p2(src, dst, cnts)

    do0 = hists[0].reshape(NB, 1)
    di0 = hists[1].reshape(NB, 1)
    do1 = hists[2].reshape(NB, 1)
    di1 = hists[3].reshape(NB, 1)

    xp = jnp.zeros((MPAD, KPAD), jnp.float32)
    xp = xp.at[:N, :IN_F].set(features)
    w1p = jnp.zeros((KPAD, HID), jnp.float32).at[:IN_F].set(W1)

    h = _mm1(xp, w1p, do0, do1)
    s1 = _p3(h, srcb, dstb, starts, tcnt, HID)

    b1t = jnp.tile(b1[None, :], (8, 1))
    g = _mm2(s1, W2, b1t, di0, di1, do0, do1)
    s2 = _p3(g, srcb, dstb, starts, tcnt, EMB)

    wcp = jnp.zeros((EMB, 128), jnp.float32).at[:, :NCLS].set(Wc)
    b2t = jnp.tile(b2[None, :], (8, 1))
    bct = jnp.tile(jnp.zeros((128,), jnp.float32).at[:NCLS].set(bc)[None, :],
                   (8, 1))
    outp = _mm3(s2, wcp, b2t, bct, di0, di1)
    return outp[:N, :NCLS]


# speedup vs baseline: 1.3055x; 1.3055x over previous
import functools

import jax
import jax.numpy as jnp
from jax import lax
from jax.experimental import pallas as pl
from jax.experimental.pallas import tpu as pltpu
from jax.experimental.pallas import tpu_sc as plsc

N = 50000
E = 800000
IN_F = 1433
HID = 384
EMB = 128
NCLS = 7

NC, NS = 2, 16
NW = NC * NS
LANES = 16

MPAD = 50176
KPAD = 1536
NB = 51200
NBINS = 392
NBINS_PAD = 400
BINSZ = 128
SENT = N
EP = 803456

W_FULL = 6250
W_BASE = W_FULL // NW
W_EXTRA = W_FULL - W_BASE * NW

_SC_MESH = dict(core_axis_name="c", subcore_axis_name="s")
_SC_PARAMS = pltpu.CompilerParams(needs_layout_passes=False)


def _worker_id():
    return lax.axis_index("s") * NC + lax.axis_index("c")


def _p1_body(src_h, dst_h, cnt_out, hist_out,
             sbuf, dbuf, ones_v, zbuf, outv, hs_src, hs_dst, cnt_s):
    cid = lax.axis_index("c")
    sid = lax.axis_index("s")
    wid = _worker_id()

    zt = NB // NS
    zv = jnp.zeros((LANES,), jnp.float32)

    def zb(i, c):
        zbuf[pl.ds(i * LANES, LANES)] = zv
        return c
    lax.fori_loop(0, zt // LANES, zb, 0)
    pltpu.sync_copy(zbuf, hs_src.at[pl.ds(sid * zt, zt)])
    pltpu.sync_copy(zbuf, hs_dst.at[pl.ds(sid * zt, zt)])

    def zc(i, c):
        cnt_s[i] = 0
        return c
    lax.fori_loop(0, NBINS_PAD, zc, 0)
    plsc.subcore_barrier()

    def ob(i, c):
        ones_v[pl.ds(i * LANES, LANES)] = jnp.ones((LANES,), jnp.float32)
        return c
    lax.fori_loop(0, BINSZ // LANES, ob, 0)

    nw = W_BASE + jnp.where(wid < W_EXTRA, 1, 0)
    wbase = wid * W_BASE + jnp.minimum(wid, W_EXTRA)

    def window(j, c):
        base = pl.multiple_of((wbase + j) * 128, 128)
        pltpu.sync_copy(src_h.at[pl.ds(base, 128)], sbuf)
        pltpu.sync_copy(dst_h.at[pl.ds(base, 128)], dbuf)
        pltpu.sync_copy(ones_v, hs_src.at[sbuf], add=True)
        pltpu.sync_copy(ones_v, hs_dst.at[dbuf], add=True)
        for g in range(8):
            dvec = dbuf[pl.ds(g * LANES, LANES)]
            bins = lax.shift_right_logical(dvec, 7)
            for l in range(LANES):
                b = bins[l]
                cnt_s[b] = cnt_s[b] + 1
        return c
    lax.fori_loop(0, nw, window, 0)

    lanes = lax.iota(jnp.int32, LANES)

    def cb(v, c):
        vec = jnp.zeros((LANES,), jnp.int32)
        for l in range(LANES):
            vec = jnp.where(lanes == l, cnt_s[v * LANES + l], vec)
        outv[pl.ds(v * LANES, LANES)] = vec
        return c
    lax.fori_loop(0, NBINS_PAD // LANES, cb, 0)
    pltpu.sync_copy(outv, cnt_out.at[wid])

    plsc.subcore_barrier()
    pltpu.sync_copy(hs_src.at[pl.ds(sid * zt, zt)],
                    hist_out.at[cid * 2].at[pl.ds(sid * zt, zt)])
    pltpu.sync_copy(hs_dst.at[pl.ds(sid * zt, zt)],
                    hist_out.at[cid * 2 + 1].at[pl.ds(sid * zt, zt)])


def _p1(src, dst):
    k = pl.kernel(
        _p1_body,
        out_type=(jax.ShapeDtypeStruct((NW, NBINS_PAD), jnp.int32),
                  jax.ShapeDtypeStruct((4, NB), jnp.float32)),
        mesh=plsc.VectorSubcoreMesh(**_SC_MESH),
        compiler_params=_SC_PARAMS,
        scratch_types=[
            pltpu.VMEM((128,), jnp.int32),
            pltpu.VMEM((128,), jnp.int32),
            pltpu.VMEM((128,), jnp.float32),
            pltpu.VMEM((NB // NS,), jnp.float32),
            pltpu.VMEM((NBINS_PAD,), jnp.int32),
            pltpu.VMEM_SHARED((NB,), jnp.float32),
            pltpu.VMEM_SHARED((NB,), jnp.float32),
            pltpu.SMEM((NBINS_PAD,), jnp.int32),
        ],
    )
    return k(src, dst)


def _p2_body(src_h, dst_h, cnt_h, srcb, dstb, starts_o, tcnt_o,
             cnt_v, s_v, t_v, sbuf, dbuf, pos_v, dstage, off_s):
    wid = _worker_id()
    pltpu.sync_copy(cnt_h, cnt_v)

    lanes = lax.iota(jnp.int32, LANES)
    zi = jnp.zeros((LANES,), jnp.int32)

    def pf(v, carry):
        colsum = zi
        mine = zi
        for w in range(NW):
            cw = cnt_v[w, pl.ds(v * LANES, LANES)]
            colsum = colsum + cw
            mine = mine + jnp.where(w < wid, cw, zi)
        t8 = (colsum + 7) & ~7
        cs = plsc.cumsum(t8)
        s_vec = cs - t8 + carry
        t_v[pl.ds(v * LANES, LANES)] = colsum
        s_v[pl.ds(v * LANES, LANES)] = s_vec
        myoff = s_vec + mine
        for l in range(LANES):
            off_s[v * LANES + l] = myoff[l]
        return carry + cs[15]
    lax.fori_loop(0, NBINS_PAD // LANES, pf, 0)

    @pl.when(wid == 0)
    def _():
        pltpu.sync_copy(s_v, starts_o)
        pltpu.sync_copy(t_v, tcnt_o)

    nw = W_BASE + jnp.where(wid < W_EXTRA, 1, 0)
    wbase = wid * W_BASE + jnp.minimum(wid, W_EXTRA)

    def window(j, c):
        base = pl.multiple_of((wbase + j) * 128, 128)
        pltpu.sync_copy(src_h.at[pl.ds(base, 128)], sbuf)
        pltpu.sync_copy(dst_h.at[pl.ds(base, 128)], dbuf)
        for g in range(8):
            dvec = dbuf[pl.ds(g * LANES, LANES)]
            bins = lax.shift_right_logical(dvec, 7)
            pos = zi
            for l in range(LANES):
                b = bins[l]
                p = off_s[b]
                off_s[b] = p + 1
                pos = jnp.where(lanes == l, p, pos)
            pos_v[pl.ds(g * LANES, LANES)] = pos
            dstage[pl.ds(g * LANES, LANES)] = dvec & 127
        pltpu.sync_copy(sbuf, srcb.at[pos_v])
        pltpu.sync_copy(dstage, dstb.at[pos_v])
        return c
    lax.fori_loop(0, nw, window, 0)


def _p2(src, dst, cnts):
    k = pl.kernel(
        _p2_body,
        out_type=(jax.ShapeDtypeStruct((EP,), jnp.int32),
                  jax.ShapeDtypeStruct((EP,), jnp.int32),
                  jax.ShapeDtypeStruct((NBINS_PAD,), jnp.int32),
                  jax.ShapeDtypeStruct((NBINS_PAD,), jnp.int32)),
        mesh=plsc.VectorSubcoreMesh(**_SC_MESH),
        compiler_params=_SC_PARAMS,
        scratch_types=[
            pltpu.VMEM((NW, NBINS_PAD), jnp.int32),
            pltpu.VMEM((NBINS_PAD,), jnp.int32),
            pltpu.VMEM((NBINS_PAD,), jnp.int32),
            pltpu.VMEM((128,), jnp.int32),
            pltpu.VMEM((128,), jnp.int32),
            pltpu.VMEM((128,), jnp.int32),
            pltpu.VMEM((128,), jnp.int32),
            pltpu.SMEM((NBINS_PAD,), jnp.int32),
        ],
    )
    return k(src, dst, cnts)


def _p3_body(F, h_hbm, srcb, dstb, starts_h, tcnt_h, out_hbm,
             sv_v, tv_v, sic, dic, si0, si1, di0, di1, rows0, rows1,
             acc_v, gsem0, gsem1, st_s, tc_s):
    wid = _worker_id()
    pltpu.sync_copy(starts_h, sv_v)
    pltpu.sync_copy(tcnt_h, tv_v)

    def ld(v, c):
        a = sv_v[pl.ds(v * LANES, LANES)]
        b = tv_v[pl.ds(v * LANES, LANES)]
        for l in range(LANES):
            st_s[v * LANES + l] = a[l]
            tc_s[v * LANES + l] = b[l]
        return c
    lax.fori_loop(0, NBINS_PAD // LANES, ld, 0)

    lanes = lax.iota(jnp.int32, LANES)
    CG = F // LANES
    zv = jnp.zeros((LANES,), jnp.float32)
    CH = 512
    WPC = CH // 64

    def clean(si_dst, di_dst, loff, rem):
        for g in range(4):
            siv = sic[pl.ds(loff + g * LANES, LANES)]
            div = dic[pl.ds(loff + g * LANES, LANES)]
            valid = (lanes + g * LANES) < rem
            si_dst[pl.ds(g * LANES, LANES)] = jnp.where(valid, siv, SENT)
            di_dst[pl.ds(g * LANES, LANES)] = jnp.where(valid, div, 0)

    def accum(di_ref, rows_ref):
        def grp(g, c2):
            dvec = di_ref[pl.ds(g * LANES, LANES)]
            for l in range(LANES):
                dl = dvec[l]
                e = g * LANES + l
                for cc in range(CG):
                    acc_v[dl, pl.ds(cc * LANES, LANES)] = (
                        acc_v[dl, pl.ds(cc * LANES, LANES)]
                        + rows_ref[e, pl.ds(cc * LANES, LANES)])
            return c2
        lax.fori_loop(0, 4, grp, 0)

    def wait0():
        pltpu.make_async_copy(h_hbm.at[si0], rows0, gsem0).wait()

    def wait1():
        pltpu.make_async_copy(h_hbm.at[si1], rows1, gsem1).wait()

    def perbin(jbin, cb):
        bin_ = jbin * NW + wid

        @pl.when(bin_ < NBINS)
        def _():
            def zrow(r, c):
                for cc in range(CG):
                    acc_v[r, pl.ds(cc * LANES, LANES)] = zv
                return c
            lax.fori_loop(0, BINSZ, zrow, 0)

            n = tc_s[bin_]
            start = pl.multiple_of(st_s[bin_], 8)
            nwin = (n + 63) // 64
            nwin2 = (nwin + 1) & ~1
            nchunk = (nwin2 + WPC - 1) // WPC

            def chunk(c, _):
                cbase = pl.multiple_of(start + c * CH, 8)
                pltpu.sync_copy(srcb.at[pl.ds(cbase, CH)], sic)
                pltpu.sync_copy(dstb.at[pl.ds(cbase, CH)], dic)
                base_w = c * WPC
                npair = jnp.minimum(WPC, nwin2 - base_w) // 2

                def pair(pp, __):
                    w0 = base_w + 2 * pp
                    clean(si0, di0, 2 * pp * 64, n - w0 * 64)
                    pltpu.async_copy(h_hbm.at[si0], rows0, gsem0)

                    @pl.when(w0 > 0)
                    def _b():
                        wait1()
                        accum(di1, rows1)
                    clean(si1, di1, (2 * pp + 1) * 64, n - (w0 + 1) * 64)
                    pltpu.async_copy(h_hbm.at[si1], rows1, gsem1)
                    wait0()
                    accum(di0, rows0)
                    return 0
                lax.fori_loop(0, npair, pair, 0)
                return 0
            lax.fori_loop(0, nchunk, chunk, 0)

            @pl.when(nwin2 > 0)
            def _d():
                wait1()
                accum(di1, rows1)
            pltpu.sync_copy(acc_v, out_hbm.at[pl.ds(bin_ * BINSZ, BINSZ)])
        return cb
    lax.fori_loop(0, 13, perbin, 0)


def _p3(h, srcb, dstb, starts, tcnt, F):
    k = pl.kernel(
        functools.partial(_p3_body, F),
        out_type=jax.ShapeDtypeStruct((MPAD, F), jnp.float32),
        mesh=plsc.VectorSubcoreMesh(**_SC_MESH),
        compiler_params=_SC_PARAMS,
        scratch_types=[
            pltpu.VMEM((NBINS_PAD,), jnp.int32),
            pltpu.VMEM((NBINS_PAD,), jnp.int32),
            pltpu.VMEM((512,), jnp.int32),
            pltpu.VMEM((512,), jnp.int32),
            pltpu.VMEM((64,), jnp.int32),
            pltpu.VMEM((64,), jnp.int32),
            pltpu.VMEM((64,), jnp.int32),
            pltpu.VMEM((64,), jnp.int32),
            pltpu.VMEM((64, F), jnp.float32),
            pltpu.VMEM((64, F), jnp.float32),
            pltpu.VMEM((BINSZ, F), jnp.float32),
            pltpu.SemaphoreType.DMA,
            pltpu.SemaphoreType.DMA,
            pltpu.SMEM((NBINS_PAD,), jnp.int32),
            pltpu.SMEM((NBINS_PAD,), jnp.int32),
        ],
    )
    return k(h, srcb, dstb, starts, tcnt)


MB = 1024
NMB = MPAD // MB


def _norm(d0, d1):
    return lax.rsqrt(jnp.maximum(d0 + d1, 1.0))


def _mm1_body(x_ref, w_ref, d0_ref, d1_ref, o_ref):
    i = pl.program_id(0)
    cid = lax.broadcasted_iota(jnp.int32, (MB, KPAD), 1)
    x = jnp.where(cid < IN_F, x_ref[...], 0.0)
    acc = jnp.dot(x, w_ref[...], preferred_element_type=jnp.float32)
    rid = lax.broadcasted_iota(jnp.int32, (MB, HID), 0) + i * MB
    o_ref[...] = jnp.where(rid < N, acc * _norm(d0_ref[...], d1_ref[...]), 0.0)


def _mm1(xp, w1p, do0, do1):
    return pl.pallas_call(
        _mm1_body,
        grid=(NMB,),
        in_specs=[
            pl.BlockSpec((MB, KPAD), lambda i: (i, 0)),
            pl.BlockSpec((KPAD, HID), lambda i: (0, 0)),
            pl.BlockSpec((MB, 1), lambda i: (i, 0)),
            pl.BlockSpec((MB, 1), lambda i: (i, 0)),
        ],
        out_specs=pl.BlockSpec((MB, HID), lambda i: (i, 0)),
        out_shape=jax.ShapeDtypeStruct((MPAD, HID), jnp.float32),
    )(xp, w1p, do0, do1)


def _mm2_body(s1_ref, w2_ref, b1_ref, di0_ref, di1_ref, do0_ref, do1_ref,
              o_ref):
    i = pl.program_id(0)
    nin = _norm(di0_ref[...], di1_ref[...])
    h1 = jnp.maximum(s1_ref[...] * nin + b1_ref[0:1, :], 0.0)
    g = jnp.dot(h1, w2_ref[...], preferred_element_type=jnp.float32)
    g = g * _norm(do0_ref[...], do1_ref[...])
    rid = lax.broadcasted_iota(jnp.int32, (MB, EMB), 0) + i * MB
    o_ref[...] = jnp.where(rid < N, g, 0.0)


def _mm2(s1, w2, b1t, di0, di1, do0, do1):
    return pl.pallas_call(
        _mm2_body,
        grid=(NMB,),
        in_specs=[
            pl.BlockSpec((MB, HID), lambda i: (i, 0)),
            pl.BlockSpec((HID, EMB), lambda i: (0, 0)),
            pl.BlockSpec((8, HID), lambda i: (0, 0)),
            pl.BlockSpec((MB, 1), lambda i: (i, 0)),
            pl.BlockSpec((MB, 1), lambda i: (i, 0)),
            pl.BlockSpec((MB, 1), lambda i: (i, 0)),
            pl.BlockSpec((MB, 1), lambda i: (i, 0)),
        ],
        out_specs=pl.BlockSpec((MB, EMB), lambda i: (i, 0)),
        out_shape=jax.ShapeDtypeStruct((MPAD, EMB), jnp.float32),
    )(s1, w2, b1t, di0, di1, do0, do1)


def _mm3_body(s2_ref, wc_ref, b2_ref, bc_ref, di0_ref, di1_ref, o_ref):
    nin = _norm(di0_ref[...], di1_ref[...])
    h2 = s2_ref[...] * nin + b2_ref[0:1, :]
    o_ref[...] = (jnp.dot(h2, wc_ref[...], preferred_element_type=jnp.float32)
                  + bc_ref[0:1, :])


def _mm3(s2, wcp, b2t, bct, di0, di1):
    return pl.pallas_call(
        _mm3_body,
        grid=(NMB,),
        in_specs=[
            pl.BlockSpec((MB, EMB), lambda i: (i, 0)),
            pl.BlockSpec((EMB, 128), lambda i: (0, 0)),
            pl.BlockSpec((8, EMB), lambda i: (0, 0)),
            pl.BlockSpec((8, 128), lambda i: (0, 0)),
            pl.BlockSpec((MB, 1), lambda i: (i, 0)),
            pl.BlockSpec((MB, 1), lambda i: (i, 0)),
        ],
        out_specs=pl.BlockSpec((MB, 128), lambda i: (i, 0)),
        out_shape=jax.ShapeDtypeStruct((MPAD, 128), jnp.float32),
    )(s2, wcp, b2t, bct, di0, di1)


def kernel(graph, features, W1, b1, W2, b2, Wc, bc):
    src = graph[0]
    dst = graph[1]

    cnts, hists = _p1(src, dst)
    srcb, dstb, starts, tcnt = _p2(src, dst, cnts)

    do0 = hists[0].reshape(NB, 1)
    di0 = hists[1].reshape(NB, 1)
    do1 = hists[2].reshape(NB, 1)
    di1 = hists[3].reshape(NB, 1)

    w1p = jnp.zeros((KPAD, HID), jnp.float32).at[:IN_F].set(W1)

    h = _mm1(features, w1p, do0, do1)
    s1 = _p3(h, srcb, dstb, starts, tcnt, HID)

    b1t = jnp.tile(b1[None, :], (8, 1))
    g = _mm2(s1, W2, b1t, di0, di1, do0, do1)
    s2 = _p3(g, srcb, dstb, starts, tcnt, EMB)

    wcp = jnp.zeros((EMB, 128), jnp.float32).at[:, :NCLS].set(Wc)
    b2t = jnp.tile(b2[None, :], (8, 1))
    bct = jnp.tile(jnp.zeros((128,), jnp.float32).at[:NCLS].set(bc)[None, :],
                   (8, 1))
    outp = _mm3(s2, wcp, b2t, bct, di0, di1)
    return outp[:N, :NCLS]
